# baseline (speedup 1.0000x reference)
import functools

import jax
import jax.numpy as jnp
from jax import lax
from jax.experimental import pallas as pl
from jax.experimental.pallas import tpu as pltpu
from jax.experimental.pallas import tpu_sc as plsc

D = 64
B = 16384
NC, NS = 2, 16
NW = NC * NS
BPW = B // NW
CH = 128
NCH = BPW // CH


def _dot_kernel(emb_hbm, vi_hbm, vj_hbm, out_hbm,
                idx_i, idx_j, rows_a, rows_b, tilebuf, out_v, sem):
    wid = lax.axis_index("s") * NC + lax.axis_index("c")
    base = wid * BPW

    idx_copies = []
    for j in range(NCH):
        idx_copies.append(pltpu.async_copy(
            vi_hbm.at[pl.ds(base + j * CH, CH)], idx_i.at[j], sem))
        idx_copies.append(pltpu.async_copy(
            vj_hbm.at[pl.ds(base + j * CH, CH)], idx_j.at[j], sem))
    for c in idx_copies:
        c.wait()

    gathers = []
    for j in range(NCH):
        gathers.append(pltpu.async_copy(
            emb_hbm.at[idx_i.at[j]], rows_a.at[pl.ds(j * CH, CH)], sem))
        gathers.append(pltpu.async_copy(
            emb_hbm.at[idx_j.at[j]], rows_b.at[pl.ds(j * CH, CH)], sem))
    for g in gathers:
        g.wait()

    iota = lax.iota(jnp.int32, 16)
    col_ids = [jnp.full((16,), c, jnp.int32) for c in range(16)]

    def block(bi, carry):
        r0 = bi * 16
        for r in range(16):
            acc = None
            for c in range(D // 16):
                a = rows_a[r0 + r, pl.ds(c * 16, 16)]
                b = rows_b[r0 + r, pl.ds(c * 16, 16)]
                acc = a * b if acc is None else acc + a * b
            tilebuf[r] = acc
        tot = None
        for c in range(16):
            g = plsc.load_gather(tilebuf, [iota, col_ids[c]])
            tot = g if tot is None else tot + g
        out_v[pl.ds(r0, 16)] = tot
        return carry

    lax.fori_loop(0, BPW // 16, block, 0)

    pltpu.sync_copy(out_v, out_hbm.at[pl.ds(base, BPW)])


@functools.partial(jax.jit, donate_argnums=())
def _run(first_emb, v_i, v_j):
    mesh = plsc.VectorSubcoreMesh(core_axis_name="c", subcore_axis_name="s")
    k = functools.partial(
        pl.kernel,
        out_type=jax.ShapeDtypeStruct((B,), jnp.float32),
        mesh=mesh,
        scratch_types=[
            pltpu.VMEM((NCH, CH), jnp.int32),
            pltpu.VMEM((NCH, CH), jnp.int32),
            pltpu.VMEM((BPW, D), jnp.float32),
            pltpu.VMEM((BPW, D), jnp.float32),
            pltpu.VMEM((16, 16), jnp.float32),
            pltpu.VMEM((BPW,), jnp.float32),
            pltpu.SemaphoreType.DMA,
        ],
        compiler_params=pltpu.CompilerParams(
            needs_layout_passes=False, use_tc_tiling_on_sc=False),
    )(_dot_kernel)
    return k(first_emb, v_i, v_j)


def kernel(v_i, v_j, first_emb, second_emb, context_emb):
    del second_emb, context_emb
    v_i = v_i.astype(jnp.int32)
    v_j = v_j.astype(jnp.int32)
    out = _run(first_emb, v_i, v_j)
    return out.reshape(B, 1)
